# Initial kernel scaffold; baseline (speedup 1.0000x reference)
import jax
import jax.numpy as jnp
from jax.experimental import pallas as pl


def kernel(flat, cu_seqlens, Wv, Wu, w_attn, Wc, bc):
    raise NotImplementedError("write your pallas kernel here")


# speedup vs baseline: 6.2431x; 6.2431x over previous
import functools

import jax
import jax.numpy as jnp
from jax.experimental import pallas as pl
from jax.experimental.pallas import tpu as pltpu


def _prism_kernel(x_ref, lo_ref, hi_ref, wvu_ref, wa_ref, wc_ref, bc_ref,
                  out_ref, acc_ref, den_ref, *, nblocks, blk, nseg, h):
    i = pl.program_id(0)

    @pl.when(i == 0)
    def _init():
        acc_ref[...] = jnp.zeros_like(acc_ref)
        den_ref[...] = jnp.zeros_like(den_ref)

    x = x_ref[...]
    g = jnp.dot(x, wvu_ref[...], preferred_element_type=jnp.float32)
    a = jnp.tanh(g[:, :h]) * jax.nn.sigmoid(g[:, h:])
    scores = jnp.dot(a, wa_ref[...], preferred_element_type=jnp.float32)

    idx = i * blk + jax.lax.broadcasted_iota(jnp.int32, (blk, 1), 0)
    oh = (idx >= lo_ref[...]) & (idx < hi_ref[...])
    e = jnp.where(oh, jnp.exp(scores), 0.0)

    acc_ref[...] += jax.lax.dot_general(
        e, x, (((0,), (0,)), ((), ())), preferred_element_type=jnp.float32)
    den_ref[...] += jax.lax.dot_general(
        e, jnp.ones((blk, 1), jnp.float32), (((0,), (0,)), ((), ())),
        preferred_element_type=jnp.float32)

    @pl.when(i == nblocks - 1)
    def _finish():
        logits = jnp.dot(acc_ref[...], wc_ref[...],
                         preferred_element_type=jnp.float32)
        out_ref[...] = logits / den_ref[...] + bc_ref[...]


def kernel(flat, cu_seqlens, Wv, Wu, w_attn, Wc, bc):
    n, d = flat.shape
    h = Wv.shape[1]
    nseg = cu_seqlens.shape[0] - 1
    c = Wc.shape[1]
    blk = 1024
    nblocks = n // blk

    wvu = jnp.concatenate([Wv, Wu], axis=1)
    cu = cu_seqlens.astype(jnp.int32)
    lo = cu[:-1].reshape(1, nseg)
    hi = cu[1:].reshape(1, nseg)
    bc2 = bc.reshape(1, c)

    grid_kernel = functools.partial(
        _prism_kernel, nblocks=nblocks, blk=blk, nseg=nseg, h=h)

    return pl.pallas_call(
        grid_kernel,
        grid=(nblocks,),
        in_specs=[
            pl.BlockSpec((blk, d), lambda i: (i, 0)),
            pl.BlockSpec((1, nseg), lambda i: (0, 0)),
            pl.BlockSpec((1, nseg), lambda i: (0, 0)),
            pl.BlockSpec((d, 2 * h), lambda i: (0, 0)),
            pl.BlockSpec((h, 1), lambda i: (0, 0)),
            pl.BlockSpec((d, c), lambda i: (0, 0)),
            pl.BlockSpec((1, c), lambda i: (0, 0)),
        ],
        out_specs=pl.BlockSpec((nseg, c), lambda i: (0, 0)),
        out_shape=jax.ShapeDtypeStruct((nseg, c), jnp.float32),
        scratch_shapes=[
            pltpu.VMEM((nseg, d), jnp.float32),
            pltpu.VMEM((nseg, 1), jnp.float32),
        ],
        compiler_params=pltpu.CompilerParams(
            dimension_semantics=("arbitrary",)),
    )(flat, lo, hi, wvu, w_attn, Wc, bc2)


# speedup vs baseline: 6.4324x; 1.0303x over previous
import functools

import jax
import jax.numpy as jnp
from jax.experimental import pallas as pl
from jax.experimental.pallas import tpu as pltpu


def _prism_kernel(x_ref, lo_ref, hi_ref, wvu_ref, wa_ref, wc_ref, bc_ref,
                  out_ref, acc_ref, den_ref, *, nblocks, blk, nseg, h):
    i = pl.program_id(0)

    @pl.when(i == 0)
    def _init():
        acc_ref[...] = jnp.zeros_like(acc_ref)
        den_ref[...] = jnp.zeros_like(den_ref)

    x = x_ref[...]
    g = jnp.dot(x.astype(jnp.bfloat16), wvu_ref[...],
                preferred_element_type=jnp.float32)
    a = jnp.tanh(g[:, :h]) * jax.nn.sigmoid(g[:, h:])
    scores = jnp.dot(a, wa_ref[...], preferred_element_type=jnp.float32)

    idx = i * blk + jax.lax.broadcasted_iota(jnp.int32, (blk, 1), 0)
    oh = (idx >= lo_ref[...]) & (idx < hi_ref[...])
    e = jnp.where(oh, jnp.exp(scores), 0.0)

    acc_ref[...] += jax.lax.dot_general(
        e, x, (((0,), (0,)), ((), ())), preferred_element_type=jnp.float32)
    den_ref[...] += jax.lax.dot_general(
        e, jnp.ones((blk, 1), jnp.float32), (((0,), (0,)), ((), ())),
        preferred_element_type=jnp.float32)

    @pl.when(i == nblocks - 1)
    def _finish():
        logits = jnp.dot(acc_ref[...], wc_ref[...],
                         preferred_element_type=jnp.float32)
        out_ref[...] = logits / den_ref[...] + bc_ref[...]


def kernel(flat, cu_seqlens, Wv, Wu, w_attn, Wc, bc):
    n, d = flat.shape
    h = Wv.shape[1]
    nseg = cu_seqlens.shape[0] - 1
    c = Wc.shape[1]
    blk = 1024
    nblocks = n // blk

    wvu = jnp.concatenate([Wv, Wu], axis=1).astype(jnp.bfloat16)
    cu = cu_seqlens.astype(jnp.int32)
    lo = cu[:-1].reshape(1, nseg)
    hi = cu[1:].reshape(1, nseg)
    bc2 = bc.reshape(1, c)

    grid_kernel = functools.partial(
        _prism_kernel, nblocks=nblocks, blk=blk, nseg=nseg, h=h)

    return pl.pallas_call(
        grid_kernel,
        grid=(nblocks,),
        in_specs=[
            pl.BlockSpec((blk, d), lambda i: (i, 0)),
            pl.BlockSpec((1, nseg), lambda i: (0, 0)),
            pl.BlockSpec((1, nseg), lambda i: (0, 0)),
            pl.BlockSpec((d, 2 * h), lambda i: (0, 0)),
            pl.BlockSpec((h, 1), lambda i: (0, 0)),
            pl.BlockSpec((d, c), lambda i: (0, 0)),
            pl.BlockSpec((1, c), lambda i: (0, 0)),
        ],
        out_specs=pl.BlockSpec((nseg, c), lambda i: (0, 0)),
        out_shape=jax.ShapeDtypeStruct((nseg, c), jnp.float32),
        scratch_shapes=[
            pltpu.VMEM((nseg, d), jnp.float32),
            pltpu.VMEM((nseg, 1), jnp.float32),
        ],
        compiler_params=pltpu.CompilerParams(
            dimension_semantics=("arbitrary",)),
    )(flat, lo, hi, wvu, w_attn, Wc, bc2)


# speedup vs baseline: 7.2321x; 1.1243x over previous
import functools

import jax
import jax.numpy as jnp
from jax.experimental import pallas as pl
from jax.experimental.pallas import tpu as pltpu


def _prism_kernel(x_ref, lo_ref, hi_ref, wvu_ref, wa_ref, wc_ref, bc_ref,
                  out_ref, acc_ref, den_ref, *, nblocks, blk, nseg, h):
    i = pl.program_id(0)

    @pl.when(i == 0)
    def _init():
        acc_ref[...] = jnp.zeros_like(acc_ref)
        den_ref[...] = jnp.zeros_like(den_ref)

    xb = x_ref[...].astype(jnp.bfloat16)
    g = jnp.dot(xb, wvu_ref[...], preferred_element_type=jnp.float32)
    a = jnp.tanh(g[:, :h]) * (0.5 * jnp.tanh(0.5 * g[:, h:]) + 0.5)
    scores = jnp.dot(a, wa_ref[...], preferred_element_type=jnp.float32)

    idx = i * blk + jax.lax.broadcasted_iota(jnp.int32, (blk, 1), 0)
    oh = (idx >= lo_ref[...]) & (idx < hi_ref[...])
    e = jnp.where(oh, jnp.exp(scores), 0.0)
    eb = e.astype(jnp.bfloat16)

    acc_ref[...] += jax.lax.dot_general(
        eb, xb, (((0,), (0,)), ((), ())), preferred_element_type=jnp.float32)
    den_ref[...] += jax.lax.dot_general(
        e, jnp.ones((blk, 1), jnp.float32), (((0,), (0,)), ((), ())),
        preferred_element_type=jnp.float32)

    @pl.when(i == nblocks - 1)
    def _finish():
        logits = jnp.dot(acc_ref[...], wc_ref[...],
                         preferred_element_type=jnp.float32)
        out_ref[...] = logits / den_ref[...] + bc_ref[...]


def kernel(flat, cu_seqlens, Wv, Wu, w_attn, Wc, bc):
    n, d = flat.shape
    h = Wv.shape[1]
    nseg = cu_seqlens.shape[0] - 1
    c = Wc.shape[1]
    blk = 2048
    nblocks = n // blk

    wvu = jnp.concatenate([Wv, Wu], axis=1).astype(jnp.bfloat16)
    cu = cu_seqlens.astype(jnp.int32)
    lo = cu[:-1].reshape(1, nseg)
    hi = cu[1:].reshape(1, nseg)
    bc2 = bc.reshape(1, c)

    grid_kernel = functools.partial(
        _prism_kernel, nblocks=nblocks, blk=blk, nseg=nseg, h=h)

    return pl.pallas_call(
        grid_kernel,
        grid=(nblocks,),
        in_specs=[
            pl.BlockSpec((blk, d), lambda i: (i, 0)),
            pl.BlockSpec((1, nseg), lambda i: (0, 0)),
            pl.BlockSpec((1, nseg), lambda i: (0, 0)),
            pl.BlockSpec((d, 2 * h), lambda i: (0, 0)),
            pl.BlockSpec((h, 1), lambda i: (0, 0)),
            pl.BlockSpec((d, c), lambda i: (0, 0)),
            pl.BlockSpec((1, c), lambda i: (0, 0)),
        ],
        out_specs=pl.BlockSpec((nseg, c), lambda i: (0, 0)),
        out_shape=jax.ShapeDtypeStruct((nseg, c), jnp.float32),
        scratch_shapes=[
            pltpu.VMEM((nseg, d), jnp.float32),
            pltpu.VMEM((nseg, 1), jnp.float32),
        ],
        compiler_params=pltpu.CompilerParams(
            dimension_semantics=("arbitrary",)),
    )(flat, lo, hi, wvu, w_attn, Wc, bc2)


# speedup vs baseline: 7.7975x; 1.0782x over previous
import functools

import jax
import jax.numpy as jnp
from jax.experimental import pallas as pl
from jax.experimental.pallas import tpu as pltpu


def _prism_kernel(x_ref, lo_ref, hi_ref, wvu_ref, wa_ref, wc_ref, bc_ref,
                  out_ref, acc_ref, den_ref, *, nblocks, blk, nseg, h):
    i = pl.program_id(0)

    @pl.when(i == 0)
    def _init():
        acc_ref[...] = jnp.zeros_like(acc_ref)
        den_ref[...] = jnp.zeros_like(den_ref)

    xb = x_ref[...].astype(jnp.bfloat16)
    x8 = xb.astype(jnp.float8_e4m3fn)
    g = jnp.dot(x8, wvu_ref[...],
                preferred_element_type=jnp.float32) * (1.0 / 64.0)
    a = jnp.tanh(g[:, :h]) * (0.5 * jnp.tanh(0.5 * g[:, h:]) + 0.5)
    scores = jnp.dot(a, wa_ref[...], preferred_element_type=jnp.float32)

    idx = i * blk + jax.lax.broadcasted_iota(jnp.int32, (blk, 1), 0)
    oh = (idx >= lo_ref[...]) & (idx < hi_ref[...])
    e = jnp.where(oh, jnp.exp(scores), 0.0)
    eb = e.astype(jnp.bfloat16)

    acc_ref[...] += jax.lax.dot_general(
        eb, xb, (((0,), (0,)), ((), ())), preferred_element_type=jnp.float32)
    den_ref[...] += jax.lax.dot_general(
        e, jnp.ones((blk, 1), jnp.float32), (((0,), (0,)), ((), ())),
        preferred_element_type=jnp.float32)

    @pl.when(i == nblocks - 1)
    def _finish():
        logits = jnp.dot(acc_ref[...], wc_ref[...],
                         preferred_element_type=jnp.float32)
        out_ref[...] = logits / den_ref[...] + bc_ref[...]


def kernel(flat, cu_seqlens, Wv, Wu, w_attn, Wc, bc):
    n, d = flat.shape
    h = Wv.shape[1]
    nseg = cu_seqlens.shape[0] - 1
    c = Wc.shape[1]
    blk = 2048
    nblocks = n // blk

    wvu = (jnp.concatenate([Wv, Wu], axis=1) * 64.0).astype(
        jnp.float8_e4m3fn)
    cu = cu_seqlens.astype(jnp.int32)
    lo = cu[:-1].reshape(1, nseg)
    hi = cu[1:].reshape(1, nseg)
    bc2 = bc.reshape(1, c)

    grid_kernel = functools.partial(
        _prism_kernel, nblocks=nblocks, blk=blk, nseg=nseg, h=h)

    return pl.pallas_call(
        grid_kernel,
        grid=(nblocks,),
        in_specs=[
            pl.BlockSpec((blk, d), lambda i: (i, 0)),
            pl.BlockSpec((1, nseg), lambda i: (0, 0)),
            pl.BlockSpec((1, nseg), lambda i: (0, 0)),
            pl.BlockSpec((d, 2 * h), lambda i: (0, 0)),
            pl.BlockSpec((h, 1), lambda i: (0, 0)),
            pl.BlockSpec((d, c), lambda i: (0, 0)),
            pl.BlockSpec((1, c), lambda i: (0, 0)),
        ],
        out_specs=pl.BlockSpec((nseg, c), lambda i: (0, 0)),
        out_shape=jax.ShapeDtypeStruct((nseg, c), jnp.float32),
        scratch_shapes=[
            pltpu.VMEM((nseg, d), jnp.float32),
            pltpu.VMEM((nseg, 1), jnp.float32),
        ],
        compiler_params=pltpu.CompilerParams(
            dimension_semantics=("arbitrary",)),
    )(flat, lo, hi, wvu, w_attn, Wc, bc2)
